# baseline (speedup 1.0000x reference)
import jax
import jax.numpy as jnp
from jax import lax
from jax.experimental import pallas as pl
from jax.experimental.pallas import tpu as pltpu
from jax.experimental.pallas import tpu_sc as plsc

GAMMA_F = 1.0
QMIN = -10.0
QMAX = 10.0
QBINS = 64
PMIN = -10.0
PMAX = 10.0
PBINS = 64
B = 4
N = 32768
NX = 128

DQ = (QMAX - QMIN) / QBINS
DP = (PMAX - PMIN) / PBINS
NBINS = QBINS * PBINS
NORM = float((GAMMA_F / jnp.pi) ** 0.25)

TOT = B * N
NTILES = 16
CHUNK = TOT // NTILES
GROUPS = CHUNK // 16
ROWS = NBINS // 16
NT1 = N // 128
TPT = CHUNK // 128


def _tables_body(xr_ref, pr_ref, pi_ref, mag_ref, sum_ref):
    x = xr_ref[...]
    pre = pr_ref[...]
    pim = pi_ref[...]

    k = lax.broadcasted_iota(jnp.int32, (1, NX), 1)
    xm = pltpu.roll(x, 1, 1)
    xp = pltpu.roll(x, NX - 1, 1)
    hi = jnp.where(k == NX - 1, x, xp)
    lo = jnp.where(k == 0, x, xm)
    w = 0.5 * (hi - lo)

    qidx = lax.broadcasted_iota(jnp.int32, (QBINS, NX), 0).astype(jnp.float32)
    qc = (qidx + 0.5) * DQ + QMIN
    d = x - qc
    m = w * jnp.exp((-0.5 * GAMMA_F) * d * d)

    pidx = lax.broadcasted_iota(jnp.int32, (PBINS, NX), 0).astype(jnp.float32)
    pc = (pidx + 0.5) * DP + PMIN
    ang = pc * x
    c = jnp.cos(ang)
    s = jnp.sin(ang)
    p1t = pre * c + pim * s
    p2t = pim * c - pre * s

    dn = (((1,), (1,)), ((), ()))
    r = lax.dot_general(m, p1t, dn, preferred_element_type=jnp.float32)
    im = lax.dot_general(m, p2t, dn, preferred_element_type=jnp.float32)

    q2 = (lax.broadcasted_iota(jnp.int32, (QBINS, PBINS), 0).astype(jnp.float32)
          + 0.5) * DQ + QMIN
    p2c = (lax.broadcasted_iota(jnp.int32, (QBINS, PBINS), 1).astype(jnp.float32)
           + 0.5) * DP + PMIN
    a2 = q2 * p2c
    cq = jnp.cos(a2)
    sq = jnp.sin(a2)

    mag_ref[...] = (NORM * NORM) * (r * r + im * im)
    sum_ref[...] = NORM * (r * (cq + sq) + im * (cq - sq))


def _compute_tables(spl_x, spl_psi_re, spl_psi_im):
    return pl.pallas_call(
        _tables_body,
        out_shape=(
            jax.ShapeDtypeStruct((QBINS, PBINS), jnp.float32),
            jax.ShapeDtypeStruct((QBINS, PBINS), jnp.float32),
        ),
    )(
        spl_x.reshape(1, NX),
        spl_psi_re.reshape(1, NX),
        spl_psi_im.reshape(1, NX),
    )


def _sc_body(trajs_hbm, fac_hbm, mag_hbm, sum_hbm, out_hbm,
             q_v, p_v, fac_v, bins_v, pres_v, sbin_v, inv_v, mag_v, sum_v,
             srow_v, u_v, partv_v, part2v_v, accv_v, outv_v,
             idx_a, idx_b, idx_sa, idx_sb,
             sem,
             pres_sh, inv_sh, u_sh, s_sh, part_sh, part2_sh):
    wid = lax.axis_index("s")
    brow = wid // 4
    boff = brow * ROWS
    base = wid * CHUNK
    qoff = brow * (4 * N) + (wid % 4) * CHUNK

    cq = pltpu.make_async_copy(trajs_hbm.at[pl.ds(qoff, CHUNK)], q_v, sem)
    cp = pltpu.make_async_copy(trajs_hbm.at[pl.ds(qoff + N, CHUNK)], p_v, sem)
    cf = pltpu.make_async_copy(fac_hbm.at[pl.ds(base, CHUNK)], fac_v, sem)
    cm = pltpu.make_async_copy(mag_hbm, mag_v, sem)
    cs = pltpu.make_async_copy(sum_hbm, sum_v, sem)
    for c in (cq, cp, cf, cm, cs):
        c.start()

    zeros16 = jnp.zeros((16,), jnp.float32)
    lane = lax.iota(jnp.int32, 16)

    def zero_body(i, _):
        pres_v[i] = zeros16
        sbin_v[i] = zeros16
        return 0

    lax.fori_loop(0, ROWS, zero_body, 0)

    def idx_body(i, _):
        v = lane + i * 16
        idx_a[pl.ds(i * 16, 16)] = v
        idx_b[pl.ds(i * 16, 16)] = v + 128
        idx_sa[pl.ds(i * 16, 16)] = v + boff
        idx_sb[pl.ds(i * 16, 16)] = v + 128 + boff
        return 0

    lax.fori_loop(0, 8, idx_body, 0)

    @pl.when(wid == 0)
    def _():
        pltpu.sync_copy(pres_v, pres_sh)
        for rr in range(B):
            pltpu.sync_copy(sbin_v, s_sh.at[pl.ds(rr * ROWS, ROWS)])

    for c in (cq, cp, cf, cm, cs):
        c.wait()

    plsc.subcore_barrier()

    ones16 = jnp.full((16,), 1.0, jnp.float32)

    def main_pass(g, acc):
        q = q_v[pl.ds(g * 16, 16)]
        p = p_v[pl.ds(g * 16, 16)]
        f = fac_v[pl.ds(g * 16, 16)]
        qb = ((q - QMIN) / DQ).astype(jnp.int32)
        pb = ((p - PMIN) / DP).astype(jnp.int32)
        qb = jnp.clip(qb, 0, QBINS - 1)
        pb = jnp.clip(pb, 0, PBINS - 1)
        bn = qb * PBINS + pb
        bins_v[pl.ds(g * 16, 16)] = bn
        hi = bn >> 4
        lo = bn & 15
        plsc.store_scatter(pres_v, [hi, lo], ones16)
        plsc.addupdate_scatter(sbin_v, [hi, lo], f)
        return acc + plsc.load_gather(mag_v, [bn])

    acc = lax.fori_loop(0, GROUPS, main_pass, zeros16)
    accv_v[...] = acc
    pltpu.sync_copy(accv_v, part_sh.at[wid])

    pltpu.sync_copy(pres_v.at[pl.ds(0, 128)], pres_sh.at[idx_a], add=True)
    pltpu.sync_copy(pres_v.at[pl.ds(128, 128)], pres_sh.at[idx_b], add=True)
    pltpu.sync_copy(sbin_v.at[pl.ds(0, 128)], s_sh.at[idx_sa], add=True)
    pltpu.sync_copy(sbin_v.at[pl.ds(128, 128)], s_sh.at[idx_sb], add=True)
    plsc.subcore_barrier()

    @pl.when(wid == 0)
    def _():
        pltpu.sync_copy(pres_sh, srow_v)

        def rank_body(i, c):
            pres = srow_v[i]
            present = pres > 0.0
            isn = present.astype(jnp.int32)
            cs = lax.cumsum(isn, axis=0)
            rk = c + cs - isn
            rk = jnp.where(present, rk, NBINS)
            plsc.store_scatter(inv_v, [rk >> 4, rk & 15], lane + i * 16)
            return c + jnp.sum(isn)

        u = lax.fori_loop(0, ROWS, rank_body, jnp.int32(0))
        u_v[...] = jnp.zeros((16,), jnp.int32) + u
        pltpu.sync_copy(inv_v.at[pl.ds(0, ROWS)], inv_sh)
        pltpu.sync_copy(u_v, u_sh)

    plsc.subcore_barrier()

    @pl.when(wid % 4 == 0)
    def _():
        pltpu.sync_copy(s_sh.at[pl.ds(boff, ROWS)], sbin_v)
        pltpu.sync_copy(inv_sh, inv_v.at[pl.ds(0, ROWS)])
        pltpu.sync_copy(u_sh, u_v)
        u = u_v[...]

        def pos_pass(i, acc2):
            jv = lane + i * 16
            iv = inv_v[i] & (NBINS - 1)
            sv = plsc.load_gather(sbin_v, [iv >> 4, iv & 15])
            sv = jnp.where(jv < u, sv, 0.0)
            bn = bins_v[pl.ds(i * 16, 16)]
            gs = plsc.load_gather(sum_v, [bn])
            return acc2 + 20000.0 * sv * sv + 200.0 * sv * gs

        acc2 = lax.fori_loop(0, ROWS, pos_pass, zeros16)
        accv_v[...] = acc2
        pltpu.sync_copy(accv_v, part2_sh.at[brow])

    plsc.subcore_barrier()

    @pl.when(wid == 0)
    def _():
        pltpu.sync_copy(part_sh, partv_v)
        pltpu.sync_copy(part2_sh, part2v_v)

        def red1(i, a):
            return a + partv_v[i]

        tot16 = lax.fori_loop(0, NTILES, red1, zeros16)

        def red2(i, a):
            return a + part2v_v[i]

        tot16 = lax.fori_loop(0, B, red2, tot16)
        total = jnp.sum(tot16)
        outv_v[...] = jnp.full((16,), 1.0, jnp.float32) * total
        pltpu.sync_copy(outv_v, out_hbm)


def _sc_main(trajs_view, fac_view, mag_flat, sum_flat):
    mesh = plsc.VectorSubcoreMesh(
        core_axis_name="c", subcore_axis_name="s", num_cores=1)
    kern = pl.kernel(
        _sc_body,
        out_type=jax.ShapeDtypeStruct((16,), jnp.float32),
        mesh=mesh,
        compiler_params=pltpu.CompilerParams(
            needs_layout_passes=False, use_tc_tiling_on_sc=False),
        scratch_types=[
            pltpu.VMEM((CHUNK,), jnp.float32),
            pltpu.VMEM((CHUNK,), jnp.float32),
            pltpu.VMEM((CHUNK,), jnp.float32),
            pltpu.VMEM((CHUNK,), jnp.int32),
            pltpu.VMEM((ROWS, 16), jnp.float32),
            pltpu.VMEM((ROWS, 16), jnp.float32),
            pltpu.VMEM((ROWS + 1, 16), jnp.int32),
            pltpu.VMEM((NBINS,), jnp.float32),
            pltpu.VMEM((NBINS,), jnp.float32),
            pltpu.VMEM((ROWS, 16), jnp.float32),
            pltpu.VMEM((16,), jnp.int32),
            pltpu.VMEM((NTILES, 16), jnp.float32),
            pltpu.VMEM((B, 16), jnp.float32),
            pltpu.VMEM((16,), jnp.float32),
            pltpu.VMEM((16,), jnp.float32),
            pltpu.VMEM((128,), jnp.int32),
            pltpu.VMEM((128,), jnp.int32),
            pltpu.VMEM((128,), jnp.int32),
            pltpu.VMEM((128,), jnp.int32),
            pltpu.SemaphoreType.DMA,
            pltpu.VMEM_SHARED((ROWS, 16), jnp.float32),
            pltpu.VMEM_SHARED((ROWS, 16), jnp.int32),
            pltpu.VMEM_SHARED((16,), jnp.int32),
            pltpu.VMEM_SHARED((B * ROWS, 16), jnp.float32),
            pltpu.VMEM_SHARED((NTILES, 16), jnp.float32),
            pltpu.VMEM_SHARED((B, 16), jnp.float32),
        ],
    )
    return kern(trajs_view, fac_view, mag_flat, sum_flat)


def kernel(factors, trajs, spl_x, spl_psi_re, spl_psi_im):
    mag, sm = _compute_tables(spl_x, spl_psi_re, spl_psi_im)
    out = _sc_main(
        trajs.transpose(0, 2, 1).reshape(-1),
        factors.reshape(-1),
        mag.reshape(-1),
        sm.reshape(-1),
    )
    return jnp.sqrt(out[0])


# speedup vs baseline: 1.1290x; 1.1290x over previous
import jax
import jax.numpy as jnp
from jax import lax
from jax.experimental import pallas as pl
from jax.experimental.pallas import tpu as pltpu
from jax.experimental.pallas import tpu_sc as plsc

GAMMA_F = 1.0
QMIN = -10.0
QMAX = 10.0
QBINS = 64
PMIN = -10.0
PMAX = 10.0
PBINS = 64
B = 4
N = 32768
NX = 128

DQ = (QMAX - QMIN) / QBINS
DP = (PMAX - PMIN) / PBINS
NBINS = QBINS * PBINS
NORM = float((GAMMA_F / jnp.pi) ** 0.25)

TOT = B * N
NTILES = 16
CHUNK = TOT // NTILES
GROUPS = CHUNK // 16
ROWS = NBINS // 16
NT1 = N // 128
TPT = CHUNK // 128


def _tables_body(xr_ref, pr_ref, pi_ref, mag_ref, sum_ref):
    x = xr_ref[...]
    pre = pr_ref[...]
    pim = pi_ref[...]

    k = lax.broadcasted_iota(jnp.int32, (1, NX), 1)
    xm = pltpu.roll(x, 1, 1)
    xp = pltpu.roll(x, NX - 1, 1)
    hi = jnp.where(k == NX - 1, x, xp)
    lo = jnp.where(k == 0, x, xm)
    w = 0.5 * (hi - lo)

    qidx = lax.broadcasted_iota(jnp.int32, (QBINS, NX), 0).astype(jnp.float32)
    qc = (qidx + 0.5) * DQ + QMIN
    d = x - qc
    m = w * jnp.exp((-0.5 * GAMMA_F) * d * d)

    pidx = lax.broadcasted_iota(jnp.int32, (PBINS, NX), 0).astype(jnp.float32)
    pc = (pidx + 0.5) * DP + PMIN
    ang = pc * x
    c = jnp.cos(ang)
    s = jnp.sin(ang)
    p1t = pre * c + pim * s
    p2t = pim * c - pre * s

    dn = (((1,), (1,)), ((), ()))
    r = lax.dot_general(m, p1t, dn, preferred_element_type=jnp.float32)
    im = lax.dot_general(m, p2t, dn, preferred_element_type=jnp.float32)

    q2 = (lax.broadcasted_iota(jnp.int32, (QBINS, PBINS), 0).astype(jnp.float32)
          + 0.5) * DQ + QMIN
    p2c = (lax.broadcasted_iota(jnp.int32, (QBINS, PBINS), 1).astype(jnp.float32)
           + 0.5) * DP + PMIN
    a2 = q2 * p2c
    cq = jnp.cos(a2)
    sq = jnp.sin(a2)

    mag_ref[...] = (NORM * NORM) * (r * r + im * im)
    sum_ref[...] = NORM * (r * (cq + sq) + im * (cq - sq))


def _compute_tables(spl_x, spl_psi_re, spl_psi_im):
    return pl.pallas_call(
        _tables_body,
        out_shape=(
            jax.ShapeDtypeStruct((QBINS, PBINS), jnp.float32),
            jax.ShapeDtypeStruct((QBINS, PBINS), jnp.float32),
        ),
    )(
        spl_x.reshape(1, NX),
        spl_psi_re.reshape(1, NX),
        spl_psi_im.reshape(1, NX),
    )


def _sc_body(trajs_hbm, fac_hbm, mag_hbm, sum_hbm, out_hbm,
             q_v, p_v, fac_v, bins_v, pres_v, sbin_v, inv_v, mag_v, sum_v,
             srow_v, u_v, partv_v, part2v_v, accv_v, outv_v,
             idx_a, idx_b, idx_sa, idx_sb,
             sem,
             pres_sh, inv_sh, u_sh, s_sh, part_sh, part2_sh):
    wid = lax.axis_index("s")
    brow = wid // 4
    boff = brow * ROWS
    nt = (wid % 4) * TPT

    cq = pltpu.make_async_copy(trajs_hbm.at[brow, pl.ds(nt, TPT), 0, :], q_v, sem)
    cp = pltpu.make_async_copy(trajs_hbm.at[brow, pl.ds(nt, TPT), 1, :], p_v, sem)
    cf = pltpu.make_async_copy(fac_hbm.at[pl.ds(nt, TPT), brow, :], fac_v, sem)
    cm = pltpu.make_async_copy(mag_hbm, mag_v, sem)
    cs = pltpu.make_async_copy(sum_hbm, sum_v, sem)
    for c in (cq, cp, cf, cm, cs):
        c.start()

    zeros16 = jnp.zeros((16,), jnp.float32)
    lane = lax.iota(jnp.int32, 16)

    def zero_body(i, _):
        pres_v[i] = zeros16
        sbin_v[i] = zeros16
        return 0

    lax.fori_loop(0, ROWS, zero_body, 0)

    def idx_body(i, _):
        v = lane + i * 16
        idx_a[pl.ds(i * 16, 16)] = v
        idx_b[pl.ds(i * 16, 16)] = v + 128
        idx_sa[pl.ds(i * 16, 16)] = v + boff
        idx_sb[pl.ds(i * 16, 16)] = v + 128 + boff
        return 0

    lax.fori_loop(0, 8, idx_body, 0)

    @pl.when(wid == 0)
    def _():
        pltpu.sync_copy(pres_v, pres_sh)
        for rr in range(B):
            pltpu.sync_copy(sbin_v, s_sh.at[pl.ds(rr * ROWS, ROWS)])

    for c in (cq, cp, cf, cm, cs):
        c.wait()

    plsc.subcore_barrier()

    ones16 = jnp.full((16,), 1.0, jnp.float32)

    def main_pass(g, acc):
        r = g >> 3
        o = (g & 7) * 16
        q = q_v[r, pl.ds(o, 16)]
        p = p_v[r, pl.ds(o, 16)]
        f = fac_v[r, pl.ds(o, 16)]
        qb = ((q - QMIN) / DQ).astype(jnp.int32)
        pb = ((p - PMIN) / DP).astype(jnp.int32)
        qb = jnp.clip(qb, 0, QBINS - 1)
        pb = jnp.clip(pb, 0, PBINS - 1)
        bn = qb * PBINS + pb
        bins_v[pl.ds(g * 16, 16)] = bn
        hi = bn >> 4
        lo = bn & 15
        plsc.store_scatter(pres_v, [hi, lo], ones16)
        plsc.addupdate_scatter(sbin_v, [hi, lo], f)
        return acc + plsc.load_gather(mag_v, [bn])

    acc = lax.fori_loop(0, GROUPS, main_pass, zeros16)
    accv_v[...] = acc
    pltpu.sync_copy(accv_v, part_sh.at[wid])

    pltpu.sync_copy(pres_v.at[pl.ds(0, 128)], pres_sh.at[idx_a], add=True)
    pltpu.sync_copy(pres_v.at[pl.ds(128, 128)], pres_sh.at[idx_b], add=True)
    pltpu.sync_copy(sbin_v.at[pl.ds(0, 128)], s_sh.at[idx_sa], add=True)
    pltpu.sync_copy(sbin_v.at[pl.ds(128, 128)], s_sh.at[idx_sb], add=True)
    plsc.subcore_barrier()

    @pl.when(wid == 0)
    def _():
        pltpu.sync_copy(pres_sh, srow_v)

        def rank_body(i, c):
            pres = srow_v[i]
            present = pres > 0.0
            isn = present.astype(jnp.int32)
            cs = lax.cumsum(isn, axis=0)
            rk = c + cs - isn
            rk = jnp.where(present, rk, NBINS)
            plsc.store_scatter(inv_v, [rk >> 4, rk & 15], lane + i * 16)
            return c + jnp.sum(isn)

        u = lax.fori_loop(0, ROWS, rank_body, jnp.int32(0))
        u_v[...] = jnp.zeros((16,), jnp.int32) + u
        pltpu.sync_copy(inv_v.at[pl.ds(0, ROWS)], inv_sh)
        pltpu.sync_copy(u_v, u_sh)

    plsc.subcore_barrier()

    @pl.when(wid % 4 == 0)
    def _():
        pltpu.sync_copy(s_sh.at[pl.ds(boff, ROWS)], sbin_v)
        pltpu.sync_copy(inv_sh, inv_v.at[pl.ds(0, ROWS)])
        pltpu.sync_copy(u_sh, u_v)
        u = u_v[...]

        def pos_pass(i, acc2):
            jv = lane + i * 16
            iv = inv_v[i] & (NBINS - 1)
            sv = plsc.load_gather(sbin_v, [iv >> 4, iv & 15])
            sv = jnp.where(jv < u, sv, 0.0)
            bn = bins_v[pl.ds(i * 16, 16)]
            gs = plsc.load_gather(sum_v, [bn])
            return acc2 + 20000.0 * sv * sv + 200.0 * sv * gs

        acc2 = lax.fori_loop(0, ROWS, pos_pass, zeros16)
        accv_v[...] = acc2
        pltpu.sync_copy(accv_v, part2_sh.at[brow])

    plsc.subcore_barrier()

    @pl.when(wid == 0)
    def _():
        pltpu.sync_copy(part_sh, partv_v)
        pltpu.sync_copy(part2_sh, part2v_v)

        def red1(i, a):
            return a + partv_v[i]

        tot16 = lax.fori_loop(0, NTILES, red1, zeros16)

        def red2(i, a):
            return a + part2v_v[i]

        tot16 = lax.fori_loop(0, B, red2, tot16)
        total = jnp.sum(tot16)
        outv_v[...] = jnp.full((16,), 1.0, jnp.float32) * total
        pltpu.sync_copy(outv_v, out_hbm)


def _sc_main(trajs_view, fac_view, mag_flat, sum_flat):
    mesh = plsc.VectorSubcoreMesh(
        core_axis_name="c", subcore_axis_name="s", num_cores=1)
    kern = pl.kernel(
        _sc_body,
        out_type=jax.ShapeDtypeStruct((16,), jnp.float32),
        mesh=mesh,
        compiler_params=pltpu.CompilerParams(
            needs_layout_passes=False, use_tc_tiling_on_sc=False),
        scratch_types=[
            pltpu.VMEM((TPT, 128), jnp.float32),
            pltpu.VMEM((TPT, 128), jnp.float32),
            pltpu.VMEM((TPT, 128), jnp.float32),
            pltpu.VMEM((CHUNK,), jnp.int32),
            pltpu.VMEM((ROWS, 16), jnp.float32),
            pltpu.VMEM((ROWS, 16), jnp.float32),
            pltpu.VMEM((ROWS + 1, 16), jnp.int32),
            pltpu.VMEM((NBINS,), jnp.float32),
            pltpu.VMEM((NBINS,), jnp.float32),
            pltpu.VMEM((ROWS, 16), jnp.float32),
            pltpu.VMEM((16,), jnp.int32),
            pltpu.VMEM((NTILES, 16), jnp.float32),
            pltpu.VMEM((B, 16), jnp.float32),
            pltpu.VMEM((16,), jnp.float32),
            pltpu.VMEM((16,), jnp.float32),
            pltpu.VMEM((128,), jnp.int32),
            pltpu.VMEM((128,), jnp.int32),
            pltpu.VMEM((128,), jnp.int32),
            pltpu.VMEM((128,), jnp.int32),
            pltpu.SemaphoreType.DMA,
            pltpu.VMEM_SHARED((ROWS, 16), jnp.float32),
            pltpu.VMEM_SHARED((ROWS, 16), jnp.int32),
            pltpu.VMEM_SHARED((16,), jnp.int32),
            pltpu.VMEM_SHARED((B * ROWS, 16), jnp.float32),
            pltpu.VMEM_SHARED((NTILES, 16), jnp.float32),
            pltpu.VMEM_SHARED((B, 16), jnp.float32),
        ],
    )
    return kern(trajs_view, fac_view, mag_flat, sum_flat)


def kernel(factors, trajs, spl_x, spl_psi_re, spl_psi_im):
    mag, sm = _compute_tables(spl_x, spl_psi_re, spl_psi_im)
    tv = trajs.reshape(B, NT1, 128, 4).transpose(0, 1, 3, 2)
    fv = factors.reshape(B, NT1, 128).transpose(1, 0, 2)
    out = _sc_main(tv, fv, mag.reshape(-1), sm.reshape(-1))
    return jnp.sqrt(out[0])


# speedup vs baseline: 1.3799x; 1.2223x over previous
import jax
import jax.numpy as jnp
from jax import lax
from jax.experimental import pallas as pl
from jax.experimental.pallas import tpu as pltpu
from jax.experimental.pallas import tpu_sc as plsc

GAMMA_F = 1.0
QMIN = -10.0
QMAX = 10.0
QBINS = 64
PMIN = -10.0
PMAX = 10.0
PBINS = 64
B = 4
N = 32768
NX = 128

DQ = (QMAX - QMIN) / QBINS
DP = (PMAX - PMIN) / PBINS
NBINS = QBINS * PBINS
NORM = float((GAMMA_F / jnp.pi) ** 0.25)

TOT = B * N
NTILES = 16
CHUNK = TOT // NTILES
GROUPS = CHUNK // 16
ROWS = NBINS // 16
NT1 = N // 128
TPT = CHUNK // 128


def _tables_body(xr_ref, pr_ref, pi_ref, mag_ref, sum_ref):
    x = xr_ref[...]
    pre = pr_ref[...]
    pim = pi_ref[...]

    k = lax.broadcasted_iota(jnp.int32, (1, NX), 1)
    xm = pltpu.roll(x, 1, 1)
    xp = pltpu.roll(x, NX - 1, 1)
    hi = jnp.where(k == NX - 1, x, xp)
    lo = jnp.where(k == 0, x, xm)
    w = 0.5 * (hi - lo)

    qidx = lax.broadcasted_iota(jnp.int32, (QBINS, NX), 0).astype(jnp.float32)
    qc = (qidx + 0.5) * DQ + QMIN
    d = x - qc
    m = w * jnp.exp((-0.5 * GAMMA_F) * d * d)

    pidx = lax.broadcasted_iota(jnp.int32, (PBINS, NX), 0).astype(jnp.float32)
    pc = (pidx + 0.5) * DP + PMIN
    ang = pc * x
    c = jnp.cos(ang)
    s = jnp.sin(ang)
    p1t = pre * c + pim * s
    p2t = pim * c - pre * s

    dn = (((1,), (1,)), ((), ()))
    r = lax.dot_general(m, p1t, dn, preferred_element_type=jnp.float32)
    im = lax.dot_general(m, p2t, dn, preferred_element_type=jnp.float32)

    q2 = (lax.broadcasted_iota(jnp.int32, (QBINS, PBINS), 0).astype(jnp.float32)
          + 0.5) * DQ + QMIN
    p2c = (lax.broadcasted_iota(jnp.int32, (QBINS, PBINS), 1).astype(jnp.float32)
           + 0.5) * DP + PMIN
    a2 = q2 * p2c
    cq = jnp.cos(a2)
    sq = jnp.sin(a2)

    mag_ref[...] = (NORM * NORM) * (r * r + im * im)
    sum_ref[...] = NORM * (r * (cq + sq) + im * (cq - sq))


def _compute_tables(spl_x, spl_psi_re, spl_psi_im):
    return pl.pallas_call(
        _tables_body,
        out_shape=(
            jax.ShapeDtypeStruct((QBINS, PBINS), jnp.float32),
            jax.ShapeDtypeStruct((QBINS, PBINS), jnp.float32),
        ),
    )(
        spl_x.reshape(1, NX),
        spl_psi_re.reshape(1, NX),
        spl_psi_im.reshape(1, NX),
    )


def _sc_body(trajs_hbm, fac_hbm, mag_hbm, sum_hbm, out_hbm,
             q_v, p_v, fac_v, bins_v, pres_v, sbin_v, inv_v, mag_v, sum_v,
             srow_v, u_v, partv_v, part2v_v, accv_v, outv_v,
             idx_a, idx_b, idx_sa, idx_sb,
             sem,
             pres_sh, inv_sh, u_sh, s_sh, part_sh, part2_sh):
    wid = lax.axis_index("s")
    brow = wid // 4
    boff = brow * ROWS
    nt = (wid % 4) * TPT

    cq = pltpu.make_async_copy(trajs_hbm.at[brow, pl.ds(nt, TPT), 0, :], q_v, sem)
    cp = pltpu.make_async_copy(trajs_hbm.at[brow, pl.ds(nt, TPT), 1, :], p_v, sem)
    cf = pltpu.make_async_copy(fac_hbm.at[pl.ds(nt, TPT), brow, :], fac_v, sem)
    cm = pltpu.make_async_copy(mag_hbm, mag_v, sem)
    cs = pltpu.make_async_copy(sum_hbm, sum_v, sem)
    for c in (cq, cp, cf, cm, cs):
        c.start()

    zeros16 = jnp.zeros((16,), jnp.float32)
    lane = lax.iota(jnp.int32, 16)

    def zero_body(i, _):
        pres_v[i] = zeros16
        sbin_v[i] = zeros16
        return 0

    lax.fori_loop(0, ROWS, zero_body, 0)

    def idx_body(i, _):
        v = lane + i * 16
        idx_a[pl.ds(i * 16, 16)] = v
        idx_b[pl.ds(i * 16, 16)] = v + 128
        idx_sa[pl.ds(i * 16, 16)] = v + boff
        idx_sb[pl.ds(i * 16, 16)] = v + 128 + boff
        return 0

    lax.fori_loop(0, 8, idx_body, 0)

    @pl.when(wid == 0)
    def _():
        pltpu.sync_copy(pres_v, pres_sh)
        for rr in range(B):
            pltpu.sync_copy(sbin_v, s_sh.at[pl.ds(rr * ROWS, ROWS)])

    for c in (cq, cp, cf, cm, cs):
        c.wait()

    plsc.subcore_barrier()

    ones16 = jnp.full((16,), 1.0, jnp.float32)

    @plsc.parallel_loop(0, GROUPS, carry=zeros16, unroll=4)
    def acc(g, a):
        r = g >> 3
        o = (g & 7) * 16
        q = q_v[r, pl.ds(o, 16)]
        p = p_v[r, pl.ds(o, 16)]
        f = fac_v[r, pl.ds(o, 16)]
        qb = (q * (1.0 / DQ) + (-QMIN / DQ)).astype(jnp.int32)
        pb = (p * (1.0 / DP) + (-PMIN / DP)).astype(jnp.int32)
        qb = jnp.clip(qb, 0, QBINS - 1)
        pb = jnp.clip(pb, 0, PBINS - 1)
        bn = qb * PBINS + pb
        bins_v[pl.ds(g * 16, 16)] = bn
        hi = bn >> 4
        lo = bn & 15
        plsc.store_scatter(pres_v, [hi, lo], ones16)
        plsc.addupdate_scatter(sbin_v, [hi, lo], f)
        return a + plsc.load_gather(mag_v, [bn])
    accv_v[...] = acc
    pltpu.sync_copy(accv_v, part_sh.at[wid])

    pltpu.sync_copy(pres_v.at[pl.ds(0, 128)], pres_sh.at[idx_a], add=True)
    pltpu.sync_copy(pres_v.at[pl.ds(128, 128)], pres_sh.at[idx_b], add=True)
    pltpu.sync_copy(sbin_v.at[pl.ds(0, 128)], s_sh.at[idx_sa], add=True)
    pltpu.sync_copy(sbin_v.at[pl.ds(128, 128)], s_sh.at[idx_sb], add=True)
    plsc.subcore_barrier()

    @pl.when(wid == 0)
    def _():
        pltpu.sync_copy(pres_sh, srow_v)

        def rank_body(i, c):
            pres = srow_v[i]
            present = pres > 0.0
            isn = present.astype(jnp.int32)
            cs = lax.cumsum(isn, axis=0)
            rk = c + cs - isn
            rk = jnp.where(present, rk, NBINS)
            plsc.store_scatter(inv_v, [rk >> 4, rk & 15], lane + i * 16)
            return c + jnp.sum(isn)

        u = lax.fori_loop(0, ROWS, rank_body, jnp.int32(0))
        u_v[...] = jnp.zeros((16,), jnp.int32) + u
        pltpu.sync_copy(inv_v.at[pl.ds(0, ROWS)], inv_sh)
        pltpu.sync_copy(u_v, u_sh)

    plsc.subcore_barrier()

    @pl.when(wid % 4 == 0)
    def _():
        pltpu.sync_copy(s_sh.at[pl.ds(boff, ROWS)], sbin_v)
        pltpu.sync_copy(inv_sh, inv_v.at[pl.ds(0, ROWS)])
        pltpu.sync_copy(u_sh, u_v)
        u = u_v[...]

        @plsc.parallel_loop(0, ROWS, carry=zeros16, unroll=4)
        def acc2(i, a2):
            jv = lane + i * 16
            iv = inv_v[i] & (NBINS - 1)
            sv = plsc.load_gather(sbin_v, [iv >> 4, iv & 15])
            sv = jnp.where(jv < u, sv, 0.0)
            bn = bins_v[pl.ds(i * 16, 16)]
            gs = plsc.load_gather(sum_v, [bn])
            return a2 + 20000.0 * sv * sv + 200.0 * sv * gs
        accv_v[...] = acc2
        pltpu.sync_copy(accv_v, part2_sh.at[brow])

    plsc.subcore_barrier()

    @pl.when(wid == 0)
    def _():
        pltpu.sync_copy(part_sh, partv_v)
        pltpu.sync_copy(part2_sh, part2v_v)

        def red1(i, a):
            return a + partv_v[i]

        tot16 = lax.fori_loop(0, NTILES, red1, zeros16)

        def red2(i, a):
            return a + part2v_v[i]

        tot16 = lax.fori_loop(0, B, red2, tot16)
        total = jnp.sum(tot16)
        outv_v[...] = jnp.full((16,), 1.0, jnp.float32) * total
        pltpu.sync_copy(outv_v, out_hbm)


def _sc_main(trajs_view, fac_view, mag_flat, sum_flat):
    mesh = plsc.VectorSubcoreMesh(
        core_axis_name="c", subcore_axis_name="s", num_cores=1)
    kern = pl.kernel(
        _sc_body,
        out_type=jax.ShapeDtypeStruct((16,), jnp.float32),
        mesh=mesh,
        compiler_params=pltpu.CompilerParams(
            needs_layout_passes=False, use_tc_tiling_on_sc=False),
        scratch_types=[
            pltpu.VMEM((TPT, 128), jnp.float32),
            pltpu.VMEM((TPT, 128), jnp.float32),
            pltpu.VMEM((TPT, 128), jnp.float32),
            pltpu.VMEM((CHUNK,), jnp.int32),
            pltpu.VMEM((ROWS, 16), jnp.float32),
            pltpu.VMEM((ROWS, 16), jnp.float32),
            pltpu.VMEM((ROWS + 1, 16), jnp.int32),
            pltpu.VMEM((NBINS,), jnp.float32),
            pltpu.VMEM((NBINS,), jnp.float32),
            pltpu.VMEM((ROWS, 16), jnp.float32),
            pltpu.VMEM((16,), jnp.int32),
            pltpu.VMEM((NTILES, 16), jnp.float32),
            pltpu.VMEM((B, 16), jnp.float32),
            pltpu.VMEM((16,), jnp.float32),
            pltpu.VMEM((16,), jnp.float32),
            pltpu.VMEM((128,), jnp.int32),
            pltpu.VMEM((128,), jnp.int32),
            pltpu.VMEM((128,), jnp.int32),
            pltpu.VMEM((128,), jnp.int32),
            pltpu.SemaphoreType.DMA,
            pltpu.VMEM_SHARED((ROWS, 16), jnp.float32),
            pltpu.VMEM_SHARED((ROWS, 16), jnp.int32),
            pltpu.VMEM_SHARED((16,), jnp.int32),
            pltpu.VMEM_SHARED((B * ROWS, 16), jnp.float32),
            pltpu.VMEM_SHARED((NTILES, 16), jnp.float32),
            pltpu.VMEM_SHARED((B, 16), jnp.float32),
        ],
    )
    return kern(trajs_view, fac_view, mag_flat, sum_flat)


def kernel(factors, trajs, spl_x, spl_psi_re, spl_psi_im):
    mag, sm = _compute_tables(spl_x, spl_psi_re, spl_psi_im)
    tv = trajs.reshape(B, NT1, 128, 4).transpose(0, 1, 3, 2)
    fv = factors.reshape(B, NT1, 128).transpose(1, 0, 2)
    out = _sc_main(tv, fv, mag.reshape(-1), sm.reshape(-1))
    return jnp.sqrt(out[0])


# speedup vs baseline: 1.4409x; 1.0442x over previous
import jax
import jax.numpy as jnp
from jax import lax
from jax.experimental import pallas as pl
from jax.experimental.pallas import tpu as pltpu
from jax.experimental.pallas import tpu_sc as plsc

GAMMA_F = 1.0
QMIN = -10.0
QMAX = 10.0
QBINS = 64
PMIN = -10.0
PMAX = 10.0
PBINS = 64
B = 4
N = 32768
NX = 128

DQ = (QMAX - QMIN) / QBINS
DP = (PMAX - PMIN) / PBINS
NBINS = QBINS * PBINS
NORM = float((GAMMA_F / jnp.pi) ** 0.25)

TOT = B * N
NTILES = 16
CHUNK = TOT // NTILES
GROUPS = CHUNK // 16
ROWS = NBINS // 16
NT1 = N // 128
TPT = CHUNK // 128


def _tables_body(xr_ref, pr_ref, pi_ref, mag_ref, sum_ref):
    x = xr_ref[...]
    pre = pr_ref[...]
    pim = pi_ref[...]

    k = lax.broadcasted_iota(jnp.int32, (1, NX), 1)
    xm = pltpu.roll(x, 1, 1)
    xp = pltpu.roll(x, NX - 1, 1)
    hi = jnp.where(k == NX - 1, x, xp)
    lo = jnp.where(k == 0, x, xm)
    w = 0.5 * (hi - lo)

    hq = QBINS // 2
    qidx = lax.broadcasted_iota(jnp.int32, (hq, NX), 0).astype(jnp.float32)
    qce = (2.0 * qidx + 0.5) * DQ + QMIN
    qco = (2.0 * qidx + 1.5) * DQ + QMIN
    de = x - qce
    do = x - qco
    me = w * jnp.exp((-0.5 * GAMMA_F) * de * de)
    mo = w * jnp.exp((-0.5 * GAMMA_F) * do * do)

    pidx = lax.broadcasted_iota(jnp.int32, (PBINS, NX), 0).astype(jnp.float32)
    pc = (pidx + 0.5) * DP + PMIN
    ang = pc * x
    c = jnp.cos(ang)
    s = jnp.sin(ang)
    p1t = pre * c + pim * s
    p2t = pim * c - pre * s

    dn = (((1,), (1,)), ((), ()))
    r = jnp.concatenate(
        [lax.dot_general(me, p1t, dn, preferred_element_type=jnp.float32),
         lax.dot_general(mo, p1t, dn, preferred_element_type=jnp.float32)],
        axis=1)
    im = jnp.concatenate(
        [lax.dot_general(me, p2t, dn, preferred_element_type=jnp.float32),
         lax.dot_general(mo, p2t, dn, preferred_element_type=jnp.float32)],
        axis=1)

    rr = lax.broadcasted_iota(jnp.int32, (hq, 2 * PBINS), 0)
    cc = lax.broadcasted_iota(jnp.int32, (hq, 2 * PBINS), 1)
    q2 = ((2 * rr + (cc >> 6)).astype(jnp.float32) + 0.5) * DQ + QMIN
    p2c = ((cc & (PBINS - 1)).astype(jnp.float32) + 0.5) * DP + PMIN
    a2 = q2 * p2c
    cq = jnp.cos(a2)
    sq = jnp.sin(a2)

    mag_ref[...] = (NORM * NORM) * (r * r + im * im)
    sum_ref[...] = NORM * (r * (cq + sq) + im * (cq - sq))


def _compute_tables(spl_x, spl_psi_re, spl_psi_im):
    return pl.pallas_call(
        _tables_body,
        out_shape=(
            jax.ShapeDtypeStruct((QBINS // 2, 2 * PBINS), jnp.float32),
            jax.ShapeDtypeStruct((QBINS // 2, 2 * PBINS), jnp.float32),
        ),
    )(
        spl_x.reshape(1, NX),
        spl_psi_re.reshape(1, NX),
        spl_psi_im.reshape(1, NX),
    )


def _sc_body(trajs_hbm, fac_hbm, mag_hbm, sum_hbm, out_hbm,
             q_v, p_v, fac_v, bins_v, pres_v, sbin_v, inv_v, mag_v, sum_v,
             srow_v, u_v, partv_v, part2v_v, accv_v, outv_v,
             idx_a, idx_b, idx_sa, idx_sb,
             sem,
             pres_sh, inv_sh, u_sh, s_sh, part_sh, part2_sh):
    wid = lax.axis_index("s")
    brow = wid // 4
    boff = brow * ROWS
    nt = (wid % 4) * TPT

    cq = pltpu.make_async_copy(trajs_hbm.at[brow, pl.ds(nt, TPT), 0, :], q_v, sem)
    cp = pltpu.make_async_copy(trajs_hbm.at[brow, pl.ds(nt, TPT), 1, :], p_v, sem)
    cf = pltpu.make_async_copy(fac_hbm.at[pl.ds(nt, TPT), brow, :], fac_v, sem)
    cm = pltpu.make_async_copy(mag_hbm, mag_v, sem)
    cs = pltpu.make_async_copy(sum_hbm, sum_v, sem)
    for c in (cq, cp, cf, cm, cs):
        c.start()

    zeros16 = jnp.zeros((16,), jnp.float32)
    lane = lax.iota(jnp.int32, 16)

    def zero_body(i, _):
        pres_v[i] = zeros16
        sbin_v[i] = zeros16
        return 0

    lax.fori_loop(0, ROWS, zero_body, 0)

    def idx_body(i, _):
        v = lane + i * 16
        idx_a[pl.ds(i * 16, 16)] = v
        idx_b[pl.ds(i * 16, 16)] = v + 128
        idx_sa[pl.ds(i * 16, 16)] = v + boff
        idx_sb[pl.ds(i * 16, 16)] = v + 128 + boff
        return 0

    lax.fori_loop(0, 8, idx_body, 0)

    @pl.when(wid == 0)
    def _():
        pltpu.sync_copy(pres_v, pres_sh)
        for rr in range(B):
            pltpu.sync_copy(sbin_v, s_sh.at[pl.ds(rr * ROWS, ROWS)])

    for c in (cq, cp, cf, cm, cs):
        c.wait()

    plsc.subcore_barrier()

    ones16 = jnp.full((16,), 1.0, jnp.float32)

    @plsc.parallel_loop(0, GROUPS, carry=zeros16, unroll=4)
    def acc(g, a):
        r = g >> 3
        o = (g & 7) * 16
        q = q_v[r, pl.ds(o, 16)]
        p = p_v[r, pl.ds(o, 16)]
        f = fac_v[r, pl.ds(o, 16)]
        qb = (q * (1.0 / DQ) + (-QMIN / DQ)).astype(jnp.int32)
        pb = (p * (1.0 / DP) + (-PMIN / DP)).astype(jnp.int32)
        qb = jnp.clip(qb, 0, QBINS - 1)
        pb = jnp.clip(pb, 0, PBINS - 1)
        bn = qb * PBINS + pb
        bins_v[pl.ds(g * 16, 16)] = bn
        hi = bn >> 4
        lo = bn & 15
        plsc.store_scatter(pres_v, [hi, lo], ones16)
        plsc.addupdate_scatter(sbin_v, [hi, lo], f)
        return a + plsc.load_gather(mag_v, [bn])
    accv_v[...] = acc
    pltpu.sync_copy(accv_v, part_sh.at[wid])

    pltpu.sync_copy(pres_v.at[pl.ds(0, 128)], pres_sh.at[idx_a], add=True)
    pltpu.sync_copy(pres_v.at[pl.ds(128, 128)], pres_sh.at[idx_b], add=True)
    pltpu.sync_copy(sbin_v.at[pl.ds(0, 128)], s_sh.at[idx_sa], add=True)
    pltpu.sync_copy(sbin_v.at[pl.ds(128, 128)], s_sh.at[idx_sb], add=True)
    plsc.subcore_barrier()

    @pl.when(wid == 0)
    def _():
        pltpu.sync_copy(pres_sh, srow_v)

        def rank_body(i, c):
            pres = srow_v[i]
            present = pres > 0.0
            isn = present.astype(jnp.int32)
            cs = lax.cumsum(isn, axis=0)
            rk = c + cs - isn
            rk = jnp.where(present, rk, NBINS)
            plsc.store_scatter(inv_v, [rk >> 4, rk & 15], lane + i * 16)
            return c + jnp.sum(isn)

        u = lax.fori_loop(0, ROWS, rank_body, jnp.int32(0))
        u_v[...] = jnp.zeros((16,), jnp.int32) + u
        pltpu.sync_copy(inv_v.at[pl.ds(0, ROWS)], inv_sh)
        pltpu.sync_copy(u_v, u_sh)

    plsc.subcore_barrier()

    @pl.when(wid % 4 == 0)
    def _():
        pltpu.sync_copy(s_sh.at[pl.ds(boff, ROWS)], sbin_v)
        pltpu.sync_copy(inv_sh, inv_v.at[pl.ds(0, ROWS)])
        pltpu.sync_copy(u_sh, u_v)
        u = u_v[...]

        @plsc.parallel_loop(0, ROWS, carry=zeros16, unroll=4)
        def acc2(i, a2):
            jv = lane + i * 16
            iv = inv_v[i] & (NBINS - 1)
            sv = plsc.load_gather(sbin_v, [iv >> 4, iv & 15])
            sv = jnp.where(jv < u, sv, 0.0)
            bn = bins_v[pl.ds(i * 16, 16)]
            gs = plsc.load_gather(sum_v, [bn])
            return a2 + 20000.0 * sv * sv + 200.0 * sv * gs
        accv_v[...] = acc2
        pltpu.sync_copy(accv_v, part2_sh.at[brow])

    plsc.subcore_barrier()

    @pl.when(wid == 0)
    def _():
        pltpu.sync_copy(part_sh, partv_v)
        pltpu.sync_copy(part2_sh, part2v_v)

        def red1(i, a):
            return a + partv_v[i]

        tot16 = lax.fori_loop(0, NTILES, red1, zeros16)

        def red2(i, a):
            return a + part2v_v[i]

        tot16 = lax.fori_loop(0, B, red2, tot16)
        total = jnp.sum(tot16)
        outv_v[...] = jnp.full((16,), 1.0, jnp.float32) * total
        pltpu.sync_copy(outv_v, out_hbm)


def _sc_main(trajs_view, fac_view, mag_flat, sum_flat):
    mesh = plsc.VectorSubcoreMesh(
        core_axis_name="c", subcore_axis_name="s", num_cores=1)
    kern = pl.kernel(
        _sc_body,
        out_type=jax.ShapeDtypeStruct((16,), jnp.float32),
        mesh=mesh,
        compiler_params=pltpu.CompilerParams(
            needs_layout_passes=False, use_tc_tiling_on_sc=False),
        scratch_types=[
            pltpu.VMEM((TPT, 128), jnp.float32),
            pltpu.VMEM((TPT, 128), jnp.float32),
            pltpu.VMEM((TPT, 128), jnp.float32),
            pltpu.VMEM((CHUNK,), jnp.int32),
            pltpu.VMEM((ROWS, 16), jnp.float32),
            pltpu.VMEM((ROWS, 16), jnp.float32),
            pltpu.VMEM((ROWS + 1, 16), jnp.int32),
            pltpu.VMEM((NBINS,), jnp.float32),
            pltpu.VMEM((NBINS,), jnp.float32),
            pltpu.VMEM((ROWS, 16), jnp.float32),
            pltpu.VMEM((16,), jnp.int32),
            pltpu.VMEM((NTILES, 16), jnp.float32),
            pltpu.VMEM((B, 16), jnp.float32),
            pltpu.VMEM((16,), jnp.float32),
            pltpu.VMEM((16,), jnp.float32),
            pltpu.VMEM((128,), jnp.int32),
            pltpu.VMEM((128,), jnp.int32),
            pltpu.VMEM((128,), jnp.int32),
            pltpu.VMEM((128,), jnp.int32),
            pltpu.SemaphoreType.DMA,
            pltpu.VMEM_SHARED((ROWS, 16), jnp.float32),
            pltpu.VMEM_SHARED((ROWS, 16), jnp.int32),
            pltpu.VMEM_SHARED((16,), jnp.int32),
            pltpu.VMEM_SHARED((B * ROWS, 16), jnp.float32),
            pltpu.VMEM_SHARED((NTILES, 16), jnp.float32),
            pltpu.VMEM_SHARED((B, 16), jnp.float32),
        ],
    )
    return kern(trajs_view, fac_view, mag_flat, sum_flat)


def kernel(factors, trajs, spl_x, spl_psi_re, spl_psi_im):
    mag, sm = _compute_tables(spl_x, spl_psi_re, spl_psi_im)
    tv = trajs.reshape(B, NT1, 128, 4).transpose(0, 1, 3, 2)
    fv = factors.reshape(B, NT1, 128).transpose(1, 0, 2)
    out = _sc_main(tv, fv, mag.reshape(-1), sm.reshape(-1))
    return jnp.sqrt(out[0])
